# Initial kernel scaffold; baseline (speedup 1.0000x reference)
import functools

import jax
import jax.numpy as jnp
from jax import lax
from jax.experimental import pallas as pl
from jax.experimental.pallas import tpu as pltpu
from jax.experimental.pallas import tpu_sc as plsc

NC = 2
NS = 16
NW = NC * NS
K = 128
NEG_SLOPE = 0.01


def _cdiv(a, b):
    return (a + b - 1) // b


def _chunks(total, step):
    out = []
    base = 0
    while base < total:
        out.append((base, min(step, total - base)))
        base += step
    return out


@functools.lru_cache(maxsize=None)
def _deg_kernel(n, nch):
    n_acc = n + 16
    zpt = n_acc // NS
    rpt = n // NS
    mesh = plsc.VectorSubcoreMesh(core_axis_name="c", subcore_axis_name="s")

    @functools.partial(
        pl.kernel,
        mesh=mesh,
        out_type=jax.ShapeDtypeStruct((NC, n, 16), jnp.float32),
        scratch_types=[
            pltpu.VMEM((nch, K), jnp.int32),
            pltpu.VMEM((K, 16), jnp.float32),
            pltpu.VMEM((K, 16), jnp.float32),
            pltpu.SemaphoreType.DMA,
            pltpu.VMEM_SHARED((n_acc, 16), jnp.float32),
        ],
    )
    def deg(dsts_hbm, out_hbm, dst_idx, ones_v, zbuf, sem, acc_sh):
        core = lax.axis_index("c")
        sid = lax.axis_index("s")
        wid = sid * NC + core
        one_row = jnp.full((16,), 1.0, jnp.float32)
        zero_row = jnp.zeros((16,), jnp.float32)
        for i in range(K):
            ones_v[i, :] = one_row
            zbuf[i, :] = zero_row
        r0z = sid * zpt
        for base, sz in _chunks(zpt, K):
            pltpu.sync_copy(zbuf.at[pl.ds(0, sz)],
                            acc_sh.at[pl.ds(r0z + base, sz)])
        pltpu.sync_copy(dsts_hbm.at[wid], dst_idx)
        plsc.subcore_barrier()

        def fire(j, _):
            pltpu.async_copy(ones_v, acc_sh.at[dst_idx.at[j]], sem, add=True)
            return 0

        lax.fori_loop(0, nch, fire, 0)

        def drain(j, _):
            pltpu.make_async_copy(ones_v, acc_sh.at[dst_idx.at[0]], sem).wait()
            return 0

        lax.fori_loop(0, nch, drain, 0)
        plsc.subcore_barrier()
        r0 = sid * rpt
        for base, sz in _chunks(rpt, K):
            pltpu.sync_copy(acc_sh.at[pl.ds(r0 + base, sz)],
                            zbuf.at[pl.ds(0, sz)])
            pltpu.sync_copy(zbuf.at[pl.ds(0, sz)],
                            out_hbm.at[core, pl.ds(r0 + base, sz)])

    return deg


@functools.lru_cache(maxsize=None)
def _agg_kernel(n, d, nch):
    n_acc = n + 16
    rpt = n // NS
    mesh = plsc.VectorSubcoreMesh(core_axis_name="c", subcore_axis_name="s")

    @functools.partial(
        pl.kernel,
        mesh=mesh,
        out_type=jax.ShapeDtypeStruct((NC, n, d), jnp.float32),
        scratch_types=[
            pltpu.VMEM((nch, K), jnp.int32),
            pltpu.VMEM((nch, K), jnp.int32),
            pltpu.VMEM((K, d), jnp.float32),
            pltpu.VMEM((K, d), jnp.float32),
            pltpu.SemaphoreType.DMA,
            pltpu.SemaphoreType.DMA,
            pltpu.VMEM_SHARED((n_acc, d), jnp.float32),
        ],
    )
    def agg(g_hbm, zeros_hbm, srcs_hbm, dsts_hbm, out_hbm,
            src_idx, dst_idx, rows0, rows1, sem0, sem1, acc_sh):
        core = lax.axis_index("c")
        sid = lax.axis_index("s")
        wid = sid * NC + core
        pltpu.sync_copy(srcs_hbm.at[wid], src_idx)
        pltpu.sync_copy(dsts_hbm.at[wid], dst_idx)
        r0 = sid * rpt
        for base, sz in _chunks(rpt, K):
            @pl.when(core == 0)
            def _(base=base, sz=sz):
                pltpu.sync_copy(g_hbm.at[pl.ds(r0 + base, sz)],
                                rows0.at[pl.ds(0, sz)])
                pltpu.sync_copy(rows0.at[pl.ds(0, sz)],
                                acc_sh.at[pl.ds(r0 + base, sz)])

            @pl.when(core == 1)
            def _(base=base, sz=sz):
                pltpu.sync_copy(zeros_hbm.at[pl.ds(r0 + base, sz)],
                                rows0.at[pl.ds(0, sz)])
                pltpu.sync_copy(rows0.at[pl.ds(0, sz)],
                                acc_sh.at[pl.ds(r0 + base, sz)])
        plsc.subcore_barrier()

        pltpu.async_copy(g_hbm.at[src_idx.at[0]], rows0, sem0)
        pltpu.async_copy(g_hbm.at[src_idx.at[1]], rows1, sem1)

        def step(jj, _):
            j0 = jj * 2
            pltpu.make_async_copy(g_hbm.at[src_idx.at[j0]], rows0, sem0).wait()
            pltpu.sync_copy(rows0, acc_sh.at[dst_idx.at[j0]], add=True)

            @pl.when(j0 + 2 < nch)
            def _():
                pltpu.async_copy(g_hbm.at[src_idx.at[j0 + 2]], rows0, sem0)

            j1 = j0 + 1
            pltpu.make_async_copy(g_hbm.at[src_idx.at[j1]], rows1, sem1).wait()
            pltpu.sync_copy(rows1, acc_sh.at[dst_idx.at[j1]], add=True)

            @pl.when(j1 + 2 < nch)
            def _():
                pltpu.async_copy(g_hbm.at[src_idx.at[j1 + 2]], rows1, sem1)

            return 0

        lax.fori_loop(0, nch // 2, step, 0)
        plsc.subcore_barrier()
        for base, sz in _chunks(rpt, K):
            pltpu.sync_copy(acc_sh.at[pl.ds(r0 + base, sz)],
                            rows0.at[pl.ds(0, sz)])
            pltpu.sync_copy(rows0.at[pl.ds(0, sz)],
                            out_hbm.at[core, pl.ds(r0 + base, sz)])

    return agg


def _leaky(t):
    return jnp.where(t >= 0, t, NEG_SLOPE * t)


def _dot(a, b):
    return jnp.dot(a, b, preferred_element_type=jnp.float32,
                   precision=lax.Precision.HIGHEST)


def _tc_first(x, w1, deg, blk=1000):
    n, d = x.shape

    def body(x_ref, w_ref, deg_ref, g_ref, dinv_ref):
        dcnt = deg_ref[0, :, 0:1] + deg_ref[1, :, 0:1] + 1.0
        dinv = jnp.broadcast_to(lax.rsqrt(dcnt), (blk, d))
        g_ref[...] = _dot(x_ref[...], w_ref[...]) * dinv
        dinv_ref[...] = dinv

    return pl.pallas_call(
        body,
        grid=(n // blk,),
        in_specs=[
            pl.BlockSpec((blk, d), lambda i: (i, 0)),
            pl.BlockSpec((d, d), lambda i: (0, 0)),
            pl.BlockSpec((NC, blk, 16), lambda i: (0, i, 0)),
        ],
        out_specs=[pl.BlockSpec((blk, d), lambda i: (i, 0))] * 2,
        out_shape=[jax.ShapeDtypeStruct((n, d), jnp.float32)] * 2,
    )(x, w1, deg)


def _tc_mid(acc, dinv, b, w, blk=1000):
    _, n, d = acc.shape

    def body(acc_ref, dinv_ref, b_ref, w_ref, g_ref):
        t = (acc_ref[0] + acc_ref[1]) * dinv_ref[...] + b_ref[...]
        g_ref[...] = _dot(_leaky(t), w_ref[...]) * dinv_ref[...]

    return pl.pallas_call(
        body,
        grid=(n // blk,),
        in_specs=[
            pl.BlockSpec((NC, blk, d), lambda i: (0, i, 0)),
            pl.BlockSpec((blk, d), lambda i: (i, 0)),
            pl.BlockSpec((1, d), lambda i: (0, 0)),
            pl.BlockSpec((d, d), lambda i: (0, 0)),
        ],
        out_specs=pl.BlockSpec((blk, d), lambda i: (i, 0)),
        out_shape=jax.ShapeDtypeStruct((n, d), jnp.float32),
    )(acc, dinv, b, w)


def _tc_last(acc, dinv, b, blk=1000):
    _, n, d = acc.shape

    def body(acc_ref, dinv_ref, b_ref, out_ref):
        t = (acc_ref[0] + acc_ref[1]) * dinv_ref[...] + b_ref[...]
        out_ref[...] = _leaky(t)

    return pl.pallas_call(
        body,
        grid=(n // blk,),
        in_specs=[
            pl.BlockSpec((NC, blk, d), lambda i: (0, i, 0)),
            pl.BlockSpec((blk, d), lambda i: (i, 0)),
            pl.BlockSpec((1, d), lambda i: (0, 0)),
        ],
        out_specs=pl.BlockSpec((blk, d), lambda i: (i, 0)),
        out_shape=jax.ShapeDtypeStruct((n, d), jnp.float32),
    )(acc, dinv, b)


def kernel(x, edge_index, W1, b1, W2, b2, W3, b3):
    n, d = x.shape
    e = edge_index.shape[1]
    src = edge_index[0].astype(jnp.int32)
    dst = edge_index[1].astype(jnp.int32)

    nch = _cdiv(_cdiv(e, NW), K)
    nch += nch % 2
    tot = NW * nch * K
    pad = tot - e
    srcp = jnp.concatenate([src, jnp.zeros((pad,), jnp.int32)]).reshape(NW, nch, K)
    dstp = jnp.concatenate([dst, jnp.full((pad,), n, jnp.int32)]).reshape(NW, nch, K)
    zeros = jnp.zeros((n, d), jnp.float32)

    deg = _deg_kernel(n, nch)(dstp)
    g, dinv = _tc_first(x, W1, deg)
    agg = _agg_kernel(n, d, nch)
    a = agg(g, zeros, srcp, dstp)
    g = _tc_mid(a, dinv, b1.reshape(1, d), W2)
    a = agg(g, zeros, srcp, dstp)
    g = _tc_mid(a, dinv, b2.reshape(1, d), W3)
    a = agg(g, zeros, srcp, dstp)
    return _tc_last(a, dinv, b3.reshape(1, d))


# speedup vs baseline: 13.9157x; 13.9157x over previous
import functools

import jax
import jax.numpy as jnp
from jax import lax
from jax.experimental import pallas as pl
from jax.experimental.pallas import tpu as pltpu
from jax.experimental.pallas import tpu_sc as plsc

NC = 2
NS = 16
KS = 128
ROW_PAD = 1280
NEG_SLOPE = 0.01


def _cdiv(a, b):
    return (a + b - 1) // b


def _chunks(total, step):
    out = []
    base = 0
    while base < total:
        out.append((base, min(step, total - base)))
        base += step
    return out


def _pick_chunk(ept):
    for k in range(128, 0, -1):
        if ept % k == 0:
            return k
    return 1


@functools.lru_cache(maxsize=None)
def _agg_kernel(n_out, d, nch, k):
    rpt = n_out // NS
    G = next(gg for gg in (8, 4, 2)
             if nch % gg == 0 and (nch // gg) % 2 == 0)
    ng = nch // G
    mesh = plsc.VectorSubcoreMesh(core_axis_name="c", subcore_axis_name="s",
                                  num_cores=1, num_subcores=NS)

    @functools.partial(
        pl.kernel,
        mesh=mesh,
        out_type=jax.ShapeDtypeStruct((n_out, d), jnp.float32),
        scratch_types=[
            pltpu.VMEM((G, k), jnp.int32),
            pltpu.VMEM((G, k), jnp.int32),
            pltpu.VMEM((G, k), jnp.int32),
            pltpu.VMEM((G, k), jnp.int32),
            pltpu.VMEM((KS, d), jnp.float32),
            pltpu.VMEM((KS, d), jnp.float32),
            pltpu.SemaphoreType.DMA,
            pltpu.SemaphoreType.DMA,
            pltpu.SemaphoreType.DMA,
            pltpu.SemaphoreType.DMA,
            pltpu.VMEM_SHARED((n_out, d), jnp.float32),
        ],
    )
    def agg(p_hbm, srcs_hbm, dsts_hbm, out_hbm,
            si0, si1, di0, di1, rows0, rows1,
            isem0, isem1, rsem0, rsem1, acc_sh):
        sid = lax.axis_index("s")
        sbuf = (si0, si1)
        dbuf = (di0, di1)
        isem = (isem0, isem1)
        rbuf = (rows0, rows1)
        rsem = (rsem0, rsem1)
        bufk = (rows0.at[pl.ds(0, k)], rows1.at[pl.ds(0, k)])

        def idx_load(g, par):
            pltpu.async_copy(srcs_hbm.at[sid, pl.ds(g * G, G)],
                             sbuf[par], isem[par])
            pltpu.async_copy(dsts_hbm.at[sid, pl.ds(g * G, G)],
                             dbuf[par], isem[par])

        def idx_wait(g, par):
            pltpu.make_async_copy(srcs_hbm.at[sid, pl.ds(g * G, G)],
                                  sbuf[par], isem[par]).wait()
            pltpu.make_async_copy(dsts_hbm.at[sid, pl.ds(g * G, G)],
                                  dbuf[par], isem[par]).wait()

        def gather(idx_row, par):
            pltpu.async_copy(p_hbm.at[idx_row], bufk[par], rsem[par])

        def gather_wait(idx_row, par):
            pltpu.make_async_copy(p_hbm.at[idx_row], bufk[par],
                                  rsem[par]).wait()

        idx_load(0, 0)
        idx_load(1, 1)
        r0 = sid * rpt
        for base, sz in _chunks(rpt, KS):
            pltpu.sync_copy(p_hbm.at[pl.ds(r0 + base, sz)],
                            rows0.at[pl.ds(0, sz)])
            pltpu.sync_copy(rows0.at[pl.ds(0, sz)],
                            acc_sh.at[pl.ds(r0 + base, sz)])
        plsc.subcore_barrier()

        idx_wait(0, 0)
        gather(si0.at[0], 0)
        gather(si0.at[1], 1)

        def step(gp, _):
            for gpar in (0, 1):
                g = gp * 2 + gpar
                si, di = sbuf[gpar], dbuf[gpar]
                nsi = sbuf[1 - gpar]
                for jj in range(G):
                    par = jj % 2
                    gather_wait(si.at[jj], par)
                    pltpu.sync_copy(bufk[par], acc_sh.at[di.at[jj]],
                                    add=True)
                    if jj == G - 2:
                        @pl.when(g + 1 < ng)
                        def _(g=g, gpar=gpar):
                            idx_wait(g + 1, 1 - gpar)
                    if jj < G - 2:
                        gather(si.at[jj + 2], par)
                    else:
                        @pl.when(g + 1 < ng)
                        def _(jj=jj, par=par, nsi=nsi):
                            gather(nsi.at[jj + 2 - G], par)

                @pl.when(g + 2 < ng)
                def _(g=g, gpar=gpar):
                    idx_load(g + 2, gpar)
            return 0

        lax.fori_loop(0, ng // 2, step, 0)
        plsc.subcore_barrier()
        for base, sz in _chunks(rpt, KS):
            pltpu.sync_copy(acc_sh.at[pl.ds(r0 + base, sz)],
                            rows0.at[pl.ds(0, sz)])
            pltpu.sync_copy(rows0.at[pl.ds(0, sz)],
                            out_hbm.at[pl.ds(r0 + base, sz)])

    return agg


def _leaky(t):
    return jnp.where(t >= 0, t, NEG_SLOPE * t)


def _dot(a, b):
    return jnp.dot(a, b, preferred_element_type=jnp.float32,
                   precision=lax.Precision.HIGHEST)


def _tc_pack(w1, w2, w3, b1, b2, b3):
    d = w1.shape[0]

    def body(w1_r, w2_r, w3_r, b1_r, b2_r, b3_r, ws_r, bs_r):
        ws_r[0] = w1_r[...]
        ws_r[1] = w2_r[...]
        ws_r[2] = w3_r[...]
        bs_r[0] = b1_r[...].reshape(1, d)
        bs_r[1] = b2_r[...].reshape(1, d)
        bs_r[2] = b3_r[...].reshape(1, d)

    return pl.pallas_call(
        body,
        out_shape=[
            jax.ShapeDtypeStruct((3, d, d), jnp.float32),
            jax.ShapeDtypeStruct((3, 1, d), jnp.float32),
        ],
    )(w1, w2, w3, b1, b2, b3)


def _tc_ones(n_out, d, blk=ROW_PAD):
    def body(ones_ref):
        ones_ref[...] = jnp.ones((blk, d), jnp.float32)

    return pl.pallas_call(
        body,
        grid=(n_out // blk,),
        out_specs=pl.BlockSpec((blk, d), lambda i: (i, 0)),
        out_shape=jax.ShapeDtypeStruct((n_out, d), jnp.float32),
    )()


def _tc_dinv(deg, blk=ROW_PAD):
    n_out, d = deg.shape

    def body(deg_ref, dinv_ref):
        dinv_ref[...] = lax.rsqrt(deg_ref[...])

    return pl.pallas_call(
        body,
        grid=(n_out // blk,),
        in_specs=[pl.BlockSpec((blk, d), lambda i: (i, 0))],
        out_specs=pl.BlockSpec((blk, d), lambda i: (i, 0)),
        out_shape=jax.ShapeDtypeStruct((n_out, d), jnp.float32),
    )(deg)


def _tc_pre(u, dinv, w, n_out, blk=ROW_PAD):
    n, d = u.shape

    def body(u_ref, dinv_ref, w_ref, p_ref):
        p_ref[...] = _dot(u_ref[...] * dinv_ref[...], w_ref[...])

    return pl.pallas_call(
        body,
        grid=(n_out // blk,),
        in_specs=[
            pl.BlockSpec((blk, d), lambda i: (i, 0)),
            pl.BlockSpec((blk, d), lambda i: (i, 0)),
            pl.BlockSpec((d, d), lambda i: (0, 0)),
        ],
        out_specs=pl.BlockSpec((blk, d), lambda i: (i, 0)),
        out_shape=jax.ShapeDtypeStruct((n_out, d), jnp.float32),
    )(u, dinv, w)


def _tc_post(a, dinv, b, n, blk=ROW_PAD):
    n_out, d = a.shape

    def body(a_ref, dinv_ref, b_ref, u_ref):
        u_ref[...] = _leaky(a_ref[...] * dinv_ref[...] + b_ref[...])

    return pl.pallas_call(
        body,
        grid=(n_out // blk,),
        in_specs=[
            pl.BlockSpec((blk, d), lambda i: (i, 0)),
            pl.BlockSpec((blk, d), lambda i: (i, 0)),
            pl.BlockSpec((1, d), lambda i: (0, 0)),
        ],
        out_specs=pl.BlockSpec((blk, d), lambda i: (i, 0)),
        out_shape=jax.ShapeDtypeStruct((n, d), jnp.float32),
    )(a, dinv, b)


def kernel(x, edge_index, W1, b1, W2, b2, W3, b3):
    n, d = x.shape
    e = edge_index.shape[1]
    src = edge_index[0].astype(jnp.int32)
    dst = edge_index[1].astype(jnp.int32)

    n_out = _cdiv(n, ROW_PAD) * ROW_PAD

    ept = _cdiv(e, NS)
    k = _pick_chunk(ept)
    nch = ept // k
    srcp = src.reshape(NS, nch, k)
    dstp = dst.reshape(NS, nch, k)

    agg = _agg_kernel(n_out, d, nch, k)
    ones = _tc_ones(n_out, d)
    deg = agg(ones, srcp, dstp)
    dinv = _tc_dinv(deg)
    ws, bs = _tc_pack(W1, W2, W3, b1, b2, b3)

    def layer(u, wb):
        w, b = wb
        p = _tc_pre(u, dinv, w, n_out)
        a = agg(p, srcp, dstp)
        return _tc_post(a, dinv, b, n), None

    u, _ = lax.scan(layer, x, (ws, bs))
    return u


# speedup vs baseline: 15.2075x; 1.0928x over previous
import functools

import jax
import jax.numpy as jnp
from jax import lax
from jax.experimental import pallas as pl
from jax.experimental.pallas import tpu as pltpu
from jax.experimental.pallas import tpu_sc as plsc

NC = 2
NS = 16
KS = 128
ROW_PAD = 1280
NEG_SLOPE = 0.01


def _cdiv(a, b):
    return (a + b - 1) // b


def _chunks(total, step):
    out = []
    base = 0
    while base < total:
        out.append((base, min(step, total - base)))
        base += step
    return out


def _pick_chunk(ept):
    for k in range(128, 0, -1):
        if ept % k == 0:
            return k
    return 1


@functools.lru_cache(maxsize=None)
def _agg_kernel(n_out, d, nch, k):
    rpt = n_out // NS
    G = next(gg for gg in (8, 4, 2)
             if nch % gg == 0 and (nch // gg) % 2 == 0)
    ng = nch // G
    mesh = plsc.VectorSubcoreMesh(core_axis_name="c", subcore_axis_name="s",
                                  num_cores=1, num_subcores=NS)

    @functools.partial(
        pl.kernel,
        mesh=mesh,
        out_type=jax.ShapeDtypeStruct((n_out, d), jnp.float32),
        scratch_types=[
            pltpu.VMEM((G, k), jnp.int32),
            pltpu.VMEM((G, k), jnp.int32),
            pltpu.VMEM((G, k), jnp.int32),
            pltpu.VMEM((G, k), jnp.int32),
            pltpu.VMEM((KS, d), jnp.float32),
            pltpu.VMEM((KS, d), jnp.float32),
            pltpu.SemaphoreType.DMA,
            pltpu.SemaphoreType.DMA,
            pltpu.SemaphoreType.DMA,
            pltpu.SemaphoreType.DMA,
            pltpu.VMEM_SHARED((n_out, d), jnp.float32),
        ],
    )
    def agg(p_hbm, srcs_hbm, dsts_hbm, out_hbm,
            si0, si1, di0, di1, rows0, rows1,
            isem0, isem1, rsem0, rsem1, acc_sh):
        sid = lax.axis_index("s")
        sbuf = (si0, si1)
        dbuf = (di0, di1)
        isem = (isem0, isem1)
        rbuf = (rows0, rows1)
        rsem = (rsem0, rsem1)
        bufk = (rows0.at[pl.ds(0, k)], rows1.at[pl.ds(0, k)])

        def idx_load(g, par):
            pltpu.async_copy(srcs_hbm.at[sid, pl.ds(g * G, G)],
                             sbuf[par], isem[par])
            pltpu.async_copy(dsts_hbm.at[sid, pl.ds(g * G, G)],
                             dbuf[par], isem[par])

        def idx_wait(g, par):
            pltpu.make_async_copy(srcs_hbm.at[sid, pl.ds(g * G, G)],
                                  sbuf[par], isem[par]).wait()
            pltpu.make_async_copy(dsts_hbm.at[sid, pl.ds(g * G, G)],
                                  dbuf[par], isem[par]).wait()

        def gather(idx_row, par):
            pltpu.async_copy(p_hbm.at[idx_row], bufk[par], rsem[par])

        def gather_wait(idx_row, par):
            pltpu.make_async_copy(p_hbm.at[idx_row], bufk[par],
                                  rsem[par]).wait()

        idx_load(0, 0)
        idx_load(1, 1)
        r0 = sid * rpt
        for base, sz in _chunks(rpt, KS):
            pltpu.sync_copy(p_hbm.at[pl.ds(r0 + base, sz)],
                            rows0.at[pl.ds(0, sz)])
            pltpu.sync_copy(rows0.at[pl.ds(0, sz)],
                            acc_sh.at[pl.ds(r0 + base, sz)])
        plsc.subcore_barrier()

        idx_wait(0, 0)
        gather(si0.at[0], 0)
        gather(si0.at[1], 1)

        def step(gp, _):
            for gpar in (0, 1):
                g = gp * 2 + gpar
                si, di = sbuf[gpar], dbuf[gpar]
                nsi = sbuf[1 - gpar]
                for jj in range(G):
                    par = jj % 2
                    gather_wait(si.at[jj], par)
                    pltpu.sync_copy(bufk[par], acc_sh.at[di.at[jj]],
                                    add=True)
                    if jj == G - 2:
                        @pl.when(g + 1 < ng)
                        def _(g=g, gpar=gpar):
                            idx_wait(g + 1, 1 - gpar)
                    if jj < G - 2:
                        gather(si.at[jj + 2], par)
                    else:
                        @pl.when(g + 1 < ng)
                        def _(jj=jj, par=par, nsi=nsi):
                            gather(nsi.at[jj + 2 - G], par)

                @pl.when(g + 2 < ng)
                def _(g=g, gpar=gpar):
                    idx_load(g + 2, gpar)
            return 0

        lax.fori_loop(0, ng // 2, step, 0)
        plsc.subcore_barrier()
        for base, sz in _chunks(rpt, KS):
            pltpu.sync_copy(acc_sh.at[pl.ds(r0 + base, sz)],
                            rows0.at[pl.ds(0, sz)])
            pltpu.sync_copy(rows0.at[pl.ds(0, sz)],
                            out_hbm.at[pl.ds(r0 + base, sz)])

    return agg


@functools.lru_cache(maxsize=None)
def _deg_agg_kernel(n_out, d, nch, k):
    rpt = n_out // NS
    mesh = plsc.VectorSubcoreMesh(core_axis_name="c", subcore_axis_name="s",
                                  num_cores=1, num_subcores=NS)

    @functools.partial(
        pl.kernel,
        mesh=mesh,
        out_type=jax.ShapeDtypeStruct((n_out, d), jnp.float32),
        scratch_types=[
            pltpu.VMEM((nch, k), jnp.int32),
            pltpu.VMEM((KS, d), jnp.float32),
            pltpu.SemaphoreType.DMA,
            pltpu.VMEM_SHARED((n_out, d), jnp.float32),
        ],
    )
    def deg(dsts_hbm, out_hbm, dst_idx, ones_v, sem, acc_sh):
        sid = lax.axis_index("s")
        pltpu.sync_copy(dsts_hbm.at[sid], dst_idx)
        one_row = jnp.ones((16,), jnp.float32)
        for i in range(KS):
            for c in range(d // 16):
                ones_v[i, pl.ds(c * 16, 16)] = one_row
        r0 = sid * rpt
        for base, sz in _chunks(rpt, KS):
            pltpu.sync_copy(ones_v.at[pl.ds(0, sz)],
                            acc_sh.at[pl.ds(r0 + base, sz)])
        plsc.subcore_barrier()
        onek = ones_v.at[pl.ds(0, k)]

        def fire(j, _):
            pltpu.async_copy(onek, acc_sh.at[dst_idx.at[j]], sem, add=True)
            return 0

        lax.fori_loop(0, nch, fire, 0)

        def drain(j, _):
            pltpu.make_async_copy(onek, acc_sh.at[dst_idx.at[0]], sem).wait()
            return 0

        lax.fori_loop(0, nch, drain, 0)
        plsc.subcore_barrier()
        for base, sz in _chunks(rpt, KS):
            pltpu.sync_copy(acc_sh.at[pl.ds(r0 + base, sz)],
                            ones_v.at[pl.ds(0, sz)])
            pltpu.sync_copy(ones_v.at[pl.ds(0, sz)],
                            out_hbm.at[pl.ds(r0 + base, sz)])

    return deg


def _leaky(t):
    return jnp.where(t >= 0, t, NEG_SLOPE * t)


def _dot(a, b):
    return jnp.dot(a, b, preferred_element_type=jnp.float32,
                   precision=lax.Precision.HIGHEST)


def _tc_pack(w1, w2, w3, b1, b2, b3):
    d = w1.shape[0]

    def body(w1_r, w2_r, w3_r, b1_r, b2_r, b3_r, ws_r, bs_r):
        ws_r[0] = w1_r[...]
        ws_r[1] = w2_r[...]
        ws_r[2] = w3_r[...]
        bs_r[0] = b1_r[...].reshape(1, d)
        bs_r[1] = b2_r[...].reshape(1, d)
        bs_r[2] = b3_r[...].reshape(1, d)

    return pl.pallas_call(
        body,
        out_shape=[
            jax.ShapeDtypeStruct((3, d, d), jnp.float32),
            jax.ShapeDtypeStruct((3, 1, d), jnp.float32),
        ],
    )(w1, w2, w3, b1, b2, b3)


def _tc_dinv(deg, blk=ROW_PAD):
    n_out, d = deg.shape

    def body(deg_ref, dinv_ref):
        dinv_ref[...] = lax.rsqrt(deg_ref[...])

    return pl.pallas_call(
        body,
        grid=(n_out // blk,),
        in_specs=[pl.BlockSpec((blk, d), lambda i: (i, 0))],
        out_specs=pl.BlockSpec((blk, d), lambda i: (i, 0)),
        out_shape=jax.ShapeDtypeStruct((n_out, d), jnp.float32),
    )(deg)


def _tc_pre(u, dinv, w, n_out, blk=ROW_PAD):
    n, d = u.shape

    def body(u_ref, dinv_ref, w_ref, p_ref):
        p_ref[...] = _dot(u_ref[...] * dinv_ref[...], w_ref[...])

    return pl.pallas_call(
        body,
        grid=(n_out // blk,),
        in_specs=[
            pl.BlockSpec((blk, d), lambda i: (i, 0)),
            pl.BlockSpec((blk, d), lambda i: (i, 0)),
            pl.BlockSpec((d, d), lambda i: (0, 0)),
        ],
        out_specs=pl.BlockSpec((blk, d), lambda i: (i, 0)),
        out_shape=jax.ShapeDtypeStruct((n_out, d), jnp.float32),
    )(u, dinv, w)


def _tc_post(a, dinv, b, n, blk=ROW_PAD):
    n_out, d = a.shape

    def body(a_ref, dinv_ref, b_ref, u_ref):
        u_ref[...] = _leaky(a_ref[...] * dinv_ref[...] + b_ref[...])

    return pl.pallas_call(
        body,
        grid=(n_out // blk,),
        in_specs=[
            pl.BlockSpec((blk, d), lambda i: (i, 0)),
            pl.BlockSpec((blk, d), lambda i: (i, 0)),
            pl.BlockSpec((1, d), lambda i: (0, 0)),
        ],
        out_specs=pl.BlockSpec((blk, d), lambda i: (i, 0)),
        out_shape=jax.ShapeDtypeStruct((n, d), jnp.float32),
    )(a, dinv, b)


def kernel(x, edge_index, W1, b1, W2, b2, W3, b3):
    n, d = x.shape
    e = edge_index.shape[1]
    src = edge_index[0].astype(jnp.int32)
    dst = edge_index[1].astype(jnp.int32)

    n_out = _cdiv(n, ROW_PAD) * ROW_PAD

    ept = _cdiv(e, NS)
    k = _pick_chunk(ept)
    nch = ept // k
    srcp = src.reshape(NS, nch, k)
    dstp = dst.reshape(NS, nch, k)

    agg = _agg_kernel(n_out, d, nch, k)
    deg = _deg_agg_kernel(n_out, d, nch, k)(dstp)
    dinv = _tc_dinv(deg)
    ws, bs = _tc_pack(W1, W2, W3, b1, b2, b3)

    def layer(u, wb):
        w, b = wb
        p = _tc_pre(u, dinv, w, n_out)
        a = agg(p, srcp, dstp)
        return _tc_post(a, dinv, b, n), None

    u, _ = lax.scan(layer, x, (ws, bs))
    return u


# speedup vs baseline: 15.3794x; 1.0113x over previous
import functools

import jax
import jax.numpy as jnp
from jax import lax
from jax.experimental import pallas as pl
from jax.experimental.pallas import tpu as pltpu
from jax.experimental.pallas import tpu_sc as plsc

NC = 2
NS = 16
KS = 128
ROW_PAD = 1280
NEG_SLOPE = 0.01


def _cdiv(a, b):
    return (a + b - 1) // b


def _chunks(total, step):
    out = []
    base = 0
    while base < total:
        out.append((base, min(step, total - base)))
        base += step
    return out


def _pick_chunk(ept):
    for k in range(128, 0, -1):
        if ept % k == 0:
            return k
    return 1


@functools.lru_cache(maxsize=None)
def _agg_kernel(n_out, d, nch, k):
    half = n_out // NC
    rpt = half // NS
    acc_rows = half + KS
    G = next(gg for gg in (8, 4, 2)
             if nch % gg == 0 and (nch // gg) % 2 == 0)
    ng = nch // G
    mesh = plsc.VectorSubcoreMesh(core_axis_name="c", subcore_axis_name="s",
                                  num_cores=NC, num_subcores=NS)

    @functools.partial(
        pl.kernel,
        mesh=mesh,
        out_type=jax.ShapeDtypeStruct((NC, half, d), jnp.float32),
        scratch_types=[
            pltpu.VMEM((G, k), jnp.int32),
            pltpu.VMEM((G, k), jnp.int32),
            pltpu.VMEM((G, k), jnp.int32),
            pltpu.VMEM((G, k), jnp.int32),
            pltpu.VMEM((KS, d), jnp.float32),
            pltpu.VMEM((KS, d), jnp.float32),
            pltpu.SemaphoreType.DMA,
            pltpu.SemaphoreType.DMA,
            pltpu.SemaphoreType.DMA,
            pltpu.SemaphoreType.DMA,
            pltpu.VMEM_SHARED((acc_rows, d), jnp.float32),
        ],
    )
    def agg(p_hbm, srcs_hbm, dsts_hbm, out_hbm,
            si0, si1, di0, di1, rows0, rows1,
            isem0, isem1, rsem0, rsem1, acc_sh):
        core = lax.axis_index("c")
        sid = lax.axis_index("s")
        sbuf = (si0, si1)
        dbuf = (di0, di1)
        isem = (isem0, isem1)
        rbuf = (rows0, rows1)
        rsem = (rsem0, rsem1)
        bufk = (rows0.at[pl.ds(0, k)], rows1.at[pl.ds(0, k)])

        def idx_load(g, par):
            pltpu.async_copy(srcs_hbm.at[sid, pl.ds(g * G, G)],
                             sbuf[par], isem[par])
            pltpu.async_copy(dsts_hbm.at[core, sid, pl.ds(g * G, G)],
                             dbuf[par], isem[par])

        def idx_wait(g, par):
            pltpu.make_async_copy(srcs_hbm.at[sid, pl.ds(g * G, G)],
                                  sbuf[par], isem[par]).wait()
            pltpu.make_async_copy(dsts_hbm.at[core, sid, pl.ds(g * G, G)],
                                  dbuf[par], isem[par]).wait()

        def gather(idx_row, par):
            pltpu.async_copy(p_hbm.at[idx_row], bufk[par], rsem[par])

        def gather_wait(idx_row, par):
            pltpu.make_async_copy(p_hbm.at[idx_row], bufk[par],
                                  rsem[par]).wait()

        idx_load(0, 0)
        idx_load(1, 1)
        r0 = sid * rpt
        p0 = core * half + r0
        for base, sz in _chunks(rpt, KS):
            pltpu.sync_copy(p_hbm.at[pl.ds(p0 + base, sz)],
                            rows0.at[pl.ds(0, sz)])
            pltpu.sync_copy(rows0.at[pl.ds(0, sz)],
                            acc_sh.at[pl.ds(r0 + base, sz)])
        plsc.subcore_barrier()

        idx_wait(0, 0)
        gather(si0.at[0], 0)
        gather(si0.at[1], 1)

        def step(gp, _):
            for gpar in (0, 1):
                g = gp * 2 + gpar
                si, di = sbuf[gpar], dbuf[gpar]
                nsi = sbuf[1 - gpar]
                for jj in range(G):
                    par = jj % 2
                    gather_wait(si.at[jj], par)
                    pltpu.sync_copy(bufk[par], acc_sh.at[di.at[jj]],
                                    add=True)
                    if jj == G - 2:
                        @pl.when(g + 1 < ng)
                        def _(g=g, gpar=gpar):
                            idx_wait(g + 1, 1 - gpar)
                    if jj < G - 2:
                        gather(si.at[jj + 2], par)
                    else:
                        @pl.when(g + 1 < ng)
                        def _(jj=jj, par=par, nsi=nsi):
                            gather(nsi.at[jj + 2 - G], par)

                @pl.when(g + 2 < ng)
                def _(g=g, gpar=gpar):
                    idx_load(g + 2, gpar)
            return 0

        lax.fori_loop(0, ng // 2, step, 0)
        plsc.subcore_barrier()
        for base, sz in _chunks(rpt, KS):
            pltpu.sync_copy(acc_sh.at[pl.ds(r0 + base, sz)],
                            rows0.at[pl.ds(0, sz)])
            pltpu.sync_copy(rows0.at[pl.ds(0, sz)],
                            out_hbm.at[core, pl.ds(r0 + base, sz)])

    return agg


@functools.lru_cache(maxsize=None)
def _deg_agg_kernel(n_out, d, nch, k):
    half = n_out // NC
    rpt = half // NS
    acc_rows = half + KS
    mesh = plsc.VectorSubcoreMesh(core_axis_name="c", subcore_axis_name="s",
                                  num_cores=NC, num_subcores=NS)

    @functools.partial(
        pl.kernel,
        mesh=mesh,
        out_type=jax.ShapeDtypeStruct((NC, half, d), jnp.float32),
        scratch_types=[
            pltpu.VMEM((nch, k), jnp.int32),
            pltpu.VMEM((KS, d), jnp.float32),
            pltpu.SemaphoreType.DMA,
            pltpu.VMEM_SHARED((acc_rows, d), jnp.float32),
        ],
    )
    def deg(dsts_hbm, out_hbm, dst_idx, ones_v, sem, acc_sh):
        core = lax.axis_index("c")
        sid = lax.axis_index("s")
        pltpu.sync_copy(dsts_hbm.at[core, sid], dst_idx)
        one_row = jnp.ones((16,), jnp.float32)
        for i in range(KS):
            for c in range(d // 16):
                ones_v[i, pl.ds(c * 16, 16)] = one_row
        r0 = sid * rpt
        for base, sz in _chunks(rpt, KS):
            pltpu.sync_copy(ones_v.at[pl.ds(0, sz)],
                            acc_sh.at[pl.ds(r0 + base, sz)])
        plsc.subcore_barrier()
        onek = ones_v.at[pl.ds(0, k)]

        def fire(j, _):
            pltpu.async_copy(onek, acc_sh.at[dst_idx.at[j]], sem, add=True)
            return 0

        lax.fori_loop(0, nch, fire, 0)

        def drain(j, _):
            pltpu.make_async_copy(onek, acc_sh.at[dst_idx.at[0]], sem).wait()
            return 0

        lax.fori_loop(0, nch, drain, 0)
        plsc.subcore_barrier()
        for base, sz in _chunks(rpt, KS):
            pltpu.sync_copy(acc_sh.at[pl.ds(r0 + base, sz)],
                            ones_v.at[pl.ds(0, sz)])
            pltpu.sync_copy(ones_v.at[pl.ds(0, sz)],
                            out_hbm.at[core, pl.ds(r0 + base, sz)])

    return deg


def _leaky(t):
    return jnp.where(t >= 0, t, NEG_SLOPE * t)


def _dot(a, b):
    return jnp.dot(a, b, preferred_element_type=jnp.float32,
                   precision=lax.Precision.HIGHEST)


def _tc_pack(w1, w2, w3, b1, b2, b3):
    d = w1.shape[0]

    def body(w1_r, w2_r, w3_r, b1_r, b2_r, b3_r, ws_r, bs_r):
        ws_r[0] = w1_r[...]
        ws_r[1] = w2_r[...]
        ws_r[2] = w3_r[...]
        bs_r[0] = b1_r[...].reshape(1, d)
        bs_r[1] = b2_r[...].reshape(1, d)
        bs_r[2] = b3_r[...].reshape(1, d)

    return pl.pallas_call(
        body,
        out_shape=[
            jax.ShapeDtypeStruct((3, d, d), jnp.float32),
            jax.ShapeDtypeStruct((3, 1, d), jnp.float32),
        ],
    )(w1, w2, w3, b1, b2, b3)


def _tc_dst2(dst2d, n_out, blk=320):
    nr, k = dst2d.shape
    half = n_out // NC

    def body(d_ref, out_ref):
        dv = d_ref[...]
        dummy = half + (dv & 63)
        out_ref[0] = jnp.where(dv < half, dv, dummy)
        out_ref[1] = jnp.where(dv >= half, dv - half, dummy)

    return pl.pallas_call(
        body,
        grid=(_cdiv(nr, blk),),
        in_specs=[pl.BlockSpec((blk, k), lambda i: (i, 0))],
        out_specs=pl.BlockSpec((NC, blk, k), lambda i: (0, i, 0)),
        out_shape=jax.ShapeDtypeStruct((NC, nr, k), jnp.int32),
    )(dst2d)


def _tc_dinv(deg, blk=ROW_PAD):
    _, half, d = deg.shape
    n_out = half * NC
    hb = half // blk

    def body(deg_ref, dinv_ref):
        dinv_ref[...] = lax.rsqrt(deg_ref[0])

    return pl.pallas_call(
        body,
        grid=(n_out // blk,),
        in_specs=[pl.BlockSpec((1, blk, d), lambda i: (i // hb, i % hb, 0))],
        out_specs=pl.BlockSpec((blk, d), lambda i: (i, 0)),
        out_shape=jax.ShapeDtypeStruct((n_out, d), jnp.float32),
    )(deg)


def _tc_pre(u, dinv, w, n_out, blk=ROW_PAD):
    n, d = u.shape

    def body(u_ref, dinv_ref, w_ref, p_ref):
        p_ref[...] = _dot(u_ref[...] * dinv_ref[...], w_ref[...])

    return pl.pallas_call(
        body,
        grid=(n_out // blk,),
        in_specs=[
            pl.BlockSpec((blk, d), lambda i: (i, 0)),
            pl.BlockSpec((blk, d), lambda i: (i, 0)),
            pl.BlockSpec((d, d), lambda i: (0, 0)),
        ],
        out_specs=pl.BlockSpec((blk, d), lambda i: (i, 0)),
        out_shape=jax.ShapeDtypeStruct((n_out, d), jnp.float32),
    )(u, dinv, w)


def _tc_post(a, dinv, b, n, blk=ROW_PAD):
    _, half, d = a.shape
    n_out = half * NC
    hb = half // blk

    def body(a_ref, dinv_ref, b_ref, u_ref):
        u_ref[...] = _leaky(a_ref[0] * dinv_ref[...] + b_ref[...])

    return pl.pallas_call(
        body,
        grid=(n_out // blk,),
        in_specs=[
            pl.BlockSpec((1, blk, d), lambda i: (i // hb, i % hb, 0)),
            pl.BlockSpec((blk, d), lambda i: (i, 0)),
            pl.BlockSpec((1, d), lambda i: (0, 0)),
        ],
        out_specs=pl.BlockSpec((blk, d), lambda i: (i, 0)),
        out_shape=jax.ShapeDtypeStruct((n, d), jnp.float32),
    )(a, dinv, b)


def kernel(x, edge_index, W1, b1, W2, b2, W3, b3):
    n, d = x.shape
    e = edge_index.shape[1]
    src = edge_index[0].astype(jnp.int32)
    dst = edge_index[1].astype(jnp.int32)

    n_out = _cdiv(n, NC * ROW_PAD) * NC * ROW_PAD

    ept = _cdiv(e, NS)
    k = _pick_chunk(ept)
    nch = ept // k
    srcp = src.reshape(NS, nch, k)
    dsts2 = _tc_dst2(dst.reshape(NS * nch, k), n_out).reshape(NC, NS, nch, k)

    agg = _agg_kernel(n_out, d, nch, k)
    deg = _deg_agg_kernel(n_out, d, nch, k)(dsts2)
    dinv = _tc_dinv(deg)
    ws, bs = _tc_pack(W1, W2, W3, b1, b2, b3)

    def layer(u, wb):
        w, b = wb
        p = _tc_pre(u, dinv, w, n_out)
        a = agg(p, srcp, dsts2)
        return _tc_post(a, dinv, b, n), None

    u, _ = lax.scan(layer, x, (ws, bs))
    return u
